# Initial kernel scaffold; baseline (speedup 1.0000x reference)
import jax
import jax.numpy as jnp
from jax.experimental import pallas as pl


def kernel(key_cache, value_cache):
    raise NotImplementedError("write your pallas kernel here")


# speedup vs baseline: 1.1747x; 1.1747x over previous
import jax
import jax.numpy as jnp
from jax.experimental import pallas as pl
from jax.experimental.pallas import tpu as pltpu

S = 2048
D = 4096
K = 64
ST = 256
NS = S // ST


def _entropy_body(x_ref, ent_ref):
    x = x_ref[0]
    m = jnp.max(x, axis=-1, keepdims=True)
    e = jnp.exp(x - m)
    z = jnp.sum(e, axis=-1, keepdims=True)
    t = jnp.sum(e * (x - m), axis=-1, keepdims=True)
    ent_ref[0] = jnp.log(z) - t / z


def _topk_body(ent_ref, idx_ref):
    ent = ent_ref[...]
    nb = ent.shape[0]
    iota_s = jax.lax.broadcasted_iota(jnp.int32, (nb, S, 1), 1)
    iota_k = jax.lax.broadcasted_iota(jnp.int32, (nb, 1, K), 2)

    def body(j, carry):
        ent_c, idx_acc = carry
        mx = jnp.max(ent_c, axis=1, keepdims=True)
        cand = jnp.where(ent_c == mx, iota_s, S)
        pick = jnp.min(cand, axis=1, keepdims=True)
        idx_acc = jnp.where(iota_k == j, pick, idx_acc)
        ent_c = jnp.where(iota_s == pick, -jnp.inf, ent_c)
        return ent_c, idx_acc

    _, idx = jax.lax.fori_loop(
        0, K, body, (ent, jnp.zeros((nb, 1, K), jnp.int32)))
    idx_ref[...] = idx


def _merge_body(idx_ref, x_hbm, x_ref, out_ref, anc_ref, acc_ref, den_ref,
                sem):
    b = pl.program_id(0)
    s = pl.program_id(1)

    @pl.when(s == 0)
    def _():
        def start(j, _):
            pltpu.make_async_copy(
                x_hbm.at[b, pl.ds(idx_ref[b, 0, j], 1), :],
                anc_ref.at[pl.ds(j, 1), :], sem).start()
            return 0

        jax.lax.fori_loop(0, K, start, 0)

        def wait(j, _):
            pltpu.make_async_copy(
                x_hbm.at[b, pl.ds(0, 1), :],
                anc_ref.at[pl.ds(j, 1), :], sem).wait()
            return 0

        jax.lax.fori_loop(0, K, wait, 0)
        acc_ref[...] = jnp.zeros_like(acc_ref)
        den_ref[...] = jnp.zeros_like(den_ref)

    x = x_ref[0]
    a = anc_ref[...]
    sim = jax.lax.dot_general(
        x, a, (((1,), (1,)), ((), ())),
        preferred_element_type=jnp.float32) * (1.0 / 64.0)
    mx = jnp.max(sim, axis=-1, keepdims=True)
    e = jnp.exp(sim - mx)
    z = jnp.sum(e, axis=-1, keepdims=True)
    w = e / z
    acc_ref[...] += jax.lax.dot_general(
        w, x, (((0,), (0,)), ((), ())), preferred_element_type=jnp.float32)
    den_ref[...] += jax.lax.dot_general(
        w, jnp.ones((ST, 128), jnp.float32), (((0,), (0,)), ((), ())),
        preferred_element_type=jnp.float32,
        precision=jax.lax.Precision.HIGHEST)

    @pl.when(s == NS - 1)
    def _():
        out_ref[0] = acc_ref[...] / (den_ref[:, 0:1] + 1e-6)


def _compress_one(x):
    B = x.shape[0]
    ent = pl.pallas_call(
        _entropy_body,
        grid=(B, NS),
        in_specs=[pl.BlockSpec((1, ST, D), lambda b, s: (b, s, 0))],
        out_specs=pl.BlockSpec((1, ST, 1), lambda b, s: (b, s, 0)),
        out_shape=jax.ShapeDtypeStruct((B, S, 1), jnp.float32),
        compiler_params=pltpu.CompilerParams(
            dimension_semantics=("parallel", "parallel")),
    )(x)
    idx = pl.pallas_call(
        _topk_body,
        in_specs=[pl.BlockSpec((B, S, 1), lambda: (0, 0, 0))],
        out_specs=pl.BlockSpec((B, 1, K), lambda: (0, 0, 0)),
        out_shape=jax.ShapeDtypeStruct((B, 1, K), jnp.int32),
    )(ent)
    out = pl.pallas_call(
        _merge_body,
        grid=(B, NS),
        in_specs=[
            pl.BlockSpec(memory_space=pltpu.SMEM),
            pl.BlockSpec(memory_space=pl.ANY),
            pl.BlockSpec((1, ST, D), lambda b, s: (b, s, 0)),
        ],
        out_specs=pl.BlockSpec((1, K, D), lambda b, s: (b, 0, 0)),
        out_shape=jax.ShapeDtypeStruct((B, K, D), jnp.float32),
        scratch_shapes=[
            pltpu.VMEM((K, D), jnp.float32),
            pltpu.VMEM((K, D), jnp.float32),
            pltpu.VMEM((K, 128), jnp.float32),
            pltpu.SemaphoreType.DMA,
        ],
        compiler_params=pltpu.CompilerParams(
            dimension_semantics=("parallel", "arbitrary")),
    )(idx, x, x)
    return out


@jax.jit
def _run(key_cache, value_cache):
    B, S_, H, Dh = key_cache.shape
    ck = _compress_one(key_cache.reshape(B, S_, H * Dh))
    cv = _compress_one(value_cache.reshape(B, S_, H * Dh))
    return (ck.reshape(B, K, H, Dh), cv.reshape(B, K, H, Dh))


def kernel(key_cache, value_cache):
    return _run(key_cache, value_cache)
